# Initial kernel scaffold; baseline (speedup 1.0000x reference)
import jax
import jax.numpy as jnp
from jax.experimental import pallas as pl


def kernel(node_mat, edge_mat, e_list, node_weight, edge_lay_1, root, bias):
    raise NotImplementedError("write your pallas kernel here")


# speedup vs baseline: 2.8132x; 2.8132x over previous
import functools

import jax
import jax.numpy as jnp
from jax import lax
from jax.experimental import pallas as pl
from jax.experimental.pallas import tpu as pltpu
from jax.experimental.pallas import tpu_sc as plsc

N = 10000
E = 160000
D_IN = 256
D_EDGE = 16
D_OUT = 256
DH = D_OUT // 2

NS = 16
EPT = E // NS
CHUNK = 80
NCHUNK = EPT // CHUNK
RCH = 128
NFULL = N // RCH
RTAIL = N - NFULL * RCH


def _node_body(x_ref, w_ref, r_ref, b_ref, p0_ref, p1_ref, base_ref):
    x = x_ref[...]
    pot = jnp.dot(x, w_ref[...], preferred_element_type=jnp.float32)
    p0_ref[...] = pot[:, :DH]
    p1_ref[...] = pot[:, DH:]
    base_ref[...] = (
        jnp.dot(x, r_ref[...], preferred_element_type=jnp.float32) + b_ref[...]
    )


def _node_transform(node_mat, node_weight, root, bias2d):
    bm = 400
    grid = (N // bm,)
    return pl.pallas_call(
        _node_body,
        grid=grid,
        in_specs=[
            pl.BlockSpec((bm, D_IN), lambda i: (i, 0)),
            pl.BlockSpec((D_IN, D_OUT), lambda i: (0, 0)),
            pl.BlockSpec((D_IN, D_OUT), lambda i: (0, 0)),
            pl.BlockSpec((1, D_OUT), lambda i: (0, 0)),
        ],
        out_specs=[
            pl.BlockSpec((bm, DH), lambda i: (i, 0)),
            pl.BlockSpec((bm, DH), lambda i: (i, 0)),
            pl.BlockSpec((bm, D_OUT), lambda i: (i, 0)),
        ],
        out_shape=[
            jax.ShapeDtypeStruct((N, DH), jnp.float32),
            jax.ShapeDtypeStruct((N, DH), jnp.float32),
            jax.ShapeDtypeStruct((N, D_OUT), jnp.float32),
        ],
    )(node_mat, node_weight, root, bias2d)


def _edge_body(e_ref, w_ref, o_ref):
    o_ref[...] = jnp.maximum(
        jnp.dot(e_ref[...], w_ref[...], preferred_element_type=jnp.float32), 0.0
    )


def _edge_mlp(edge_mat, edge_lay_1):
    bm = 1600
    grid = (E // bm,)
    return pl.pallas_call(
        _edge_body,
        grid=grid,
        in_specs=[
            pl.BlockSpec((bm, D_EDGE), lambda i: (i, 0)),
            pl.BlockSpec((D_EDGE, D_OUT), lambda i: (0, 0)),
        ],
        out_specs=pl.BlockSpec((bm, D_OUT), lambda i: (i, 0)),
        out_shape=jax.ShapeDtypeStruct((E, D_OUT), jnp.float32),
    )(edge_mat, edge_lay_1)


def _sc_body(pot0, pot1, base, emlp, row_hbm, col_hbm, out,
             idxr, idxc, gbuf, ebuf, iobuf, accum, sem):
    c = lax.axis_index("c")
    s = lax.axis_index("s")

    def init_it(i, carry):
        j = i * NS + s

        @pl.when(j < NFULL)
        def _():
            r0 = j * RCH
            pltpu.sync_copy(base.at[pl.ds(r0, RCH), pl.ds(c * DH, DH)], iobuf)
            pltpu.sync_copy(iobuf, accum.at[pl.ds(r0, RCH)])

        return carry

    lax.fori_loop(0, (NFULL + NS - 1) // NS, init_it, 0)

    @pl.when(s == 0)
    def _():
        r0 = NFULL * RCH
        pltpu.sync_copy(
            base.at[pl.ds(r0, RTAIL), pl.ds(c * DH, DH)], iobuf.at[pl.ds(0, RTAIL)]
        )
        pltpu.sync_copy(iobuf.at[pl.ds(0, RTAIL)], accum.at[pl.ds(r0, RTAIL)])

    plsc.subcore_barrier()

    def edge_it(i, carry):
        e0 = s * EPT + i * CHUNK
        pltpu.sync_copy(row_hbm.at[pl.ds(e0, CHUNK)], idxr)
        pltpu.sync_copy(col_hbm.at[pl.ds(e0, CHUNK)], idxc)

        @pl.when(c == 0)
        def _():
            pltpu.async_copy(pot0.at[idxc], gbuf, sem).wait()

        @pl.when(c == 1)
        def _():
            pltpu.async_copy(pot1.at[idxc], gbuf, sem).wait()

        pltpu.sync_copy(gbuf, accum.at[idxr], add=True)
        pltpu.sync_copy(emlp.at[pl.ds(e0, CHUNK), pl.ds(c * DH, DH)], ebuf)
        pltpu.sync_copy(ebuf, accum.at[idxc], add=True)
        return carry

    lax.fori_loop(0, NCHUNK, edge_it, 0)
    plsc.subcore_barrier()

    def out_it(i, carry):
        j = i * NS + s

        @pl.when(j < NFULL)
        def _():
            r0 = j * RCH
            pltpu.sync_copy(accum.at[pl.ds(r0, RCH)], iobuf)
            pltpu.sync_copy(iobuf, out.at[pl.ds(r0, RCH), pl.ds(c * DH, DH)])

        return carry

    lax.fori_loop(0, (NFULL + NS - 1) // NS, out_it, 0)

    @pl.when(s == 0)
    def _():
        r0 = NFULL * RCH
        pltpu.sync_copy(accum.at[pl.ds(r0, RTAIL)], iobuf.at[pl.ds(0, RTAIL)])
        pltpu.sync_copy(
            iobuf.at[pl.ds(0, RTAIL)], out.at[pl.ds(r0, RTAIL), pl.ds(c * DH, DH)]
        )


def _sc_scatter(pot0, pot1, base, emlp, row, col):
    mesh = plsc.VectorSubcoreMesh(core_axis_name="c", subcore_axis_name="s")
    k = pl.kernel(
        _sc_body,
        mesh=mesh,
        out_type=jax.ShapeDtypeStruct((N, D_OUT), jnp.float32),
        scratch_types=[
            pltpu.VMEM((CHUNK,), jnp.int32),
            pltpu.VMEM((CHUNK,), jnp.int32),
            pltpu.VMEM((CHUNK, DH), jnp.float32),
            pltpu.VMEM((CHUNK, DH), jnp.float32),
            pltpu.VMEM((RCH, DH), jnp.float32),
            pltpu.VMEM_SHARED((N, DH), jnp.float32),
            pltpu.SemaphoreType.DMA,
        ],
    )
    return k(pot0, pot1, base, emlp, row, col)


@jax.jit
def kernel(node_mat, edge_mat, e_list, node_weight, edge_lay_1, root, bias):
    pot0, pot1, base = _node_transform(
        node_mat, node_weight, root, bias.reshape(1, D_OUT)
    )
    emlp = _edge_mlp(edge_mat, edge_lay_1)
    row = e_list[0]
    col = e_list[1]
    return _sc_scatter(pot0, pot1, base, emlp, row, col)


# speedup vs baseline: 4.0975x; 1.4565x over previous
import functools

import jax
import jax.numpy as jnp
from jax import lax
from jax.experimental import pallas as pl
from jax.experimental.pallas import tpu as pltpu
from jax.experimental.pallas import tpu_sc as plsc

N = 10000
E = 160000
D_IN = 256
D_EDGE = 16
D_OUT = 256
DH = D_OUT // 2

NS = 16
CHUNK = 80
NCHUNK = E // CHUNK
CPT = NCHUNK // NS
NB = 2
NGRP = CPT // NB
RCH = CHUNK
NRCHUNK = N // RCH


def _node_body(x_ref, w_ref, r_ref, b_ref, p0_ref, p1_ref, base_ref):
    x = x_ref[...]
    pot = jnp.dot(x, w_ref[...], preferred_element_type=jnp.float32)
    p0_ref[...] = pot[:, :DH]
    p1_ref[...] = pot[:, DH:]
    base_ref[...] = (
        jnp.dot(x, r_ref[...], preferred_element_type=jnp.float32) + b_ref[...]
    )


def _node_transform(node_mat, node_weight, root, bias2d):
    bm = 400
    grid = (N // bm,)
    return pl.pallas_call(
        _node_body,
        grid=grid,
        in_specs=[
            pl.BlockSpec((bm, D_IN), lambda i: (i, 0)),
            pl.BlockSpec((D_IN, D_OUT), lambda i: (0, 0)),
            pl.BlockSpec((D_IN, D_OUT), lambda i: (0, 0)),
            pl.BlockSpec((1, D_OUT), lambda i: (0, 0)),
        ],
        out_specs=[
            pl.BlockSpec((bm, DH), lambda i: (i, 0)),
            pl.BlockSpec((bm, DH), lambda i: (i, 0)),
            pl.BlockSpec((bm, D_OUT), lambda i: (i, 0)),
        ],
        out_shape=[
            jax.ShapeDtypeStruct((N, DH), jnp.float32),
            jax.ShapeDtypeStruct((N, DH), jnp.float32),
            jax.ShapeDtypeStruct((N, D_OUT), jnp.float32),
        ],
    )(node_mat, node_weight, root, bias2d)


def _edge_body(e_ref, w_ref, o_ref):
    o_ref[...] = jnp.maximum(
        jnp.dot(e_ref[...], w_ref[...], preferred_element_type=jnp.float32), 0.0
    )


def _edge_mlp(edge_mat, edge_lay_1):
    bm = 1600
    grid = (E // bm,)
    return pl.pallas_call(
        _edge_body,
        grid=grid,
        in_specs=[
            pl.BlockSpec((bm, D_EDGE), lambda i: (i, 0)),
            pl.BlockSpec((D_EDGE, D_OUT), lambda i: (0, 0)),
        ],
        out_specs=pl.BlockSpec((bm, D_OUT), lambda i: (i, 0)),
        out_shape=jax.ShapeDtypeStruct((E, D_OUT), jnp.float32),
    )(edge_mat, edge_lay_1)


def _sc_body(pot0, pot1, base, emlp, row_hbm, col_hbm, out,
             idxr, idxc, gbuf, ebuf, accum,
             sem, sem_i, sem_e, sem_g, sem_s):
    c = lax.axis_index("c")
    s = lax.axis_index("s")

    def init_it(i, carry):
        j = i * NS + s

        @pl.when(j < NRCHUNK)
        def _():
            r0 = j * RCH
            pltpu.sync_copy(base.at[pl.ds(r0, RCH), pl.ds(c * DH, DH)], gbuf.at[0])
            pltpu.sync_copy(gbuf.at[0], accum.at[pl.ds(r0, RCH)])

        return carry

    lax.fori_loop(0, (NRCHUNK + NS - 1) // NS, init_it, 0)
    plsc.subcore_barrier()

    def edge_grp(g, carry):
        e0s = [((g * NB + b) * NS + s) * CHUNK for b in range(NB)]
        dis, des = [], []
        for b in range(NB):
            e0 = e0s[b]
            dis.append(
                pltpu.async_copy(row_hbm.at[pl.ds(e0, CHUNK)], idxr.at[b], sem_i))
            dis.append(
                pltpu.async_copy(col_hbm.at[pl.ds(e0, CHUNK)], idxc.at[b], sem_i))
            des.append(pltpu.async_copy(
                emlp.at[pl.ds(e0, CHUNK), pl.ds(c * DH, DH)], ebuf.at[b], sem_e))
        for d in dis:
            d.wait()
        for b in range(NB):
            @pl.when(c == 0)
            def _(b=b):
                pltpu.async_copy(pot0.at[idxc.at[b]], gbuf.at[b], sem_g)

            @pl.when(c == 1)
            def _(b=b):
                pltpu.async_copy(pot1.at[idxc.at[b]], gbuf.at[b], sem_g)

        for b in range(NB):
            pltpu.make_async_copy(pot0.at[idxc.at[b]], gbuf.at[b], sem_g).wait()
        for d in des:
            d.wait()
        dss = []
        for b in range(NB):
            dss.append(pltpu.async_copy(
                gbuf.at[b], accum.at[idxr.at[b]], sem_s, add=True))
            dss.append(pltpu.async_copy(
                ebuf.at[b], accum.at[idxc.at[b]], sem_s, add=True))
        for d in dss:
            d.wait()
        return carry

    lax.fori_loop(0, NGRP, edge_grp, 0)

    def _tail():
        e0 = ((NGRP * NB) * NS + s) * CHUNK
        pltpu.sync_copy(row_hbm.at[pl.ds(e0, CHUNK)], idxr.at[0])
        pltpu.sync_copy(col_hbm.at[pl.ds(e0, CHUNK)], idxc.at[0])

        @pl.when(c == 0)
        def _():
            pltpu.async_copy(pot0.at[idxc.at[0]], gbuf.at[0], sem).wait()

        @pl.when(c == 1)
        def _():
            pltpu.async_copy(pot1.at[idxc.at[0]], gbuf.at[0], sem).wait()

        pltpu.sync_copy(gbuf.at[0], accum.at[idxr.at[0]], add=True)
        pltpu.sync_copy(emlp.at[pl.ds(e0, CHUNK), pl.ds(c * DH, DH)], ebuf.at[0])
        pltpu.sync_copy(ebuf.at[0], accum.at[idxc.at[0]], add=True)

    _tail()
    plsc.subcore_barrier()

    def out_it(i, carry):
        j = i * NS + s

        @pl.when(j < NRCHUNK)
        def _():
            r0 = j * RCH
            pltpu.sync_copy(accum.at[pl.ds(r0, RCH)], gbuf.at[0])
            pltpu.sync_copy(gbuf.at[0], out.at[pl.ds(r0, RCH), pl.ds(c * DH, DH)])

        return carry

    lax.fori_loop(0, (NRCHUNK + NS - 1) // NS, out_it, 0)


def _sc_scatter(pot0, pot1, base, emlp, row, col):
    mesh = plsc.VectorSubcoreMesh(core_axis_name="c", subcore_axis_name="s")
    k = pl.kernel(
        _sc_body,
        mesh=mesh,
        out_type=jax.ShapeDtypeStruct((N, D_OUT), jnp.float32),
        scratch_types=[
            pltpu.VMEM((NB, CHUNK), jnp.int32),
            pltpu.VMEM((NB, CHUNK), jnp.int32),
            pltpu.VMEM((NB, CHUNK, DH), jnp.float32),
            pltpu.VMEM((NB, CHUNK, DH), jnp.float32),
            pltpu.VMEM_SHARED((N, DH), jnp.float32),
            pltpu.SemaphoreType.DMA,
            pltpu.SemaphoreType.DMA,
            pltpu.SemaphoreType.DMA,
            pltpu.SemaphoreType.DMA,
            pltpu.SemaphoreType.DMA,
        ],
    )
    return k(pot0, pot1, base, emlp, row, col)


@jax.jit
def kernel(node_mat, edge_mat, e_list, node_weight, edge_lay_1, root, bias):
    pot0, pot1, base = _node_transform(
        node_mat, node_weight, root, bias.reshape(1, D_OUT)
    )
    emlp = _edge_mlp(edge_mat, edge_lay_1)
    row = e_list[0]
    col = e_list[1]
    return _sc_scatter(pot0, pot1, base, emlp, row, col)


# speedup vs baseline: 4.5284x; 1.1052x over previous
import functools

import jax
import jax.numpy as jnp
from jax import lax
from jax.experimental import pallas as pl
from jax.experimental.pallas import tpu as pltpu
from jax.experimental.pallas import tpu_sc as plsc

N = 10000
E = 160000
D_IN = 256
D_EDGE = 16
D_OUT = 256
DH = D_OUT // 2

NS = 16
CHUNK = 64
NSLOT = 3
NCHUNK = E // CHUNK
CPT = NCHUNK // NS
NLEFT = NCHUNK - CPT * NS
NITER = CPT // NSLOT
RCH = CHUNK
NRFULL = N // RCH
RTAIL = N - NRFULL * RCH


def _node_body(x_ref, w_ref, r_ref, b_ref, p0_ref, p1_ref, base_ref):
    x = x_ref[...]
    pot = jnp.dot(x, w_ref[...], preferred_element_type=jnp.float32)
    p0_ref[...] = pot[:, :DH]
    p1_ref[...] = pot[:, DH:]
    base_ref[...] = (
        jnp.dot(x, r_ref[...], preferred_element_type=jnp.float32) + b_ref[...]
    )


def _node_transform(node_mat, node_weight, root, bias2d):
    bm = 400
    grid = (N // bm,)
    return pl.pallas_call(
        _node_body,
        grid=grid,
        in_specs=[
            pl.BlockSpec((bm, D_IN), lambda i: (i, 0)),
            pl.BlockSpec((D_IN, D_OUT), lambda i: (0, 0)),
            pl.BlockSpec((D_IN, D_OUT), lambda i: (0, 0)),
            pl.BlockSpec((1, D_OUT), lambda i: (0, 0)),
        ],
        out_specs=[
            pl.BlockSpec((bm, DH), lambda i: (i, 0)),
            pl.BlockSpec((bm, DH), lambda i: (i, 0)),
            pl.BlockSpec((bm, D_OUT), lambda i: (i, 0)),
        ],
        out_shape=[
            jax.ShapeDtypeStruct((N, DH), jnp.float32),
            jax.ShapeDtypeStruct((N, DH), jnp.float32),
            jax.ShapeDtypeStruct((N, D_OUT), jnp.float32),
        ],
    )(node_mat, node_weight, root, bias2d)


def _edge_body(e_ref, w_ref, o_ref):
    o_ref[...] = jnp.maximum(
        jnp.dot(e_ref[...], w_ref[...], preferred_element_type=jnp.float32), 0.0
    )


def _edge_mlp(edge_mat, edge_lay_1):
    bm = 1600
    grid = (E // bm,)
    return pl.pallas_call(
        _edge_body,
        grid=grid,
        in_specs=[
            pl.BlockSpec((bm, D_EDGE), lambda i: (i, 0)),
            pl.BlockSpec((D_EDGE, D_OUT), lambda i: (0, 0)),
        ],
        out_specs=pl.BlockSpec((bm, D_OUT), lambda i: (i, 0)),
        out_shape=jax.ShapeDtypeStruct((E, D_OUT), jnp.float32),
    )(edge_mat, edge_lay_1)


def _sc_body(pot0, pot1, base, emlp, row_hbm, col_hbm, out,
             idxr, idxc, gbuf, ebuf, accum, sem,
             si0, si1, si2, se0, se1, se2, sg0, sg1, sg2, ss0, ss1, ss2):
    c = lax.axis_index("c")
    s = lax.axis_index("s")
    si = (si0, si1, si2)
    se = (se0, se1, se2)
    sg = (sg0, sg1, sg2)
    ss = (ss0, ss1, ss2)

    def init_it(i, carry):
        j = i * NS + s

        @pl.when(j < NRFULL)
        def _():
            r0 = j * RCH
            pltpu.sync_copy(base.at[pl.ds(r0, RCH), pl.ds(c * DH, DH)], gbuf.at[0])
            pltpu.sync_copy(gbuf.at[0], accum.at[pl.ds(r0, RCH)])

        return carry

    lax.fori_loop(0, (NRFULL + NS - 1) // NS, init_it, 0)

    @pl.when(s == 0)
    def _():
        r0 = NRFULL * RCH
        pltpu.sync_copy(
            base.at[pl.ds(r0, RTAIL), pl.ds(c * DH, DH)],
            gbuf.at[0, pl.ds(0, RTAIL)],
        )
        pltpu.sync_copy(gbuf.at[0, pl.ds(0, RTAIL)], accum.at[pl.ds(r0, RTAIL)])

    plsc.subcore_barrier()

    def edge_it(t, carry):
        e0s = [((t * NSLOT + u) * NS + s) * CHUNK for u in range(NSLOT)]
        for u in range(NSLOT):
            @pl.when(t > 0)
            def _(u=u):
                pltpu.make_async_copy(
                    gbuf.at[u], accum.at[idxr.at[u]], ss[u]).wait()
                pltpu.make_async_copy(
                    ebuf.at[u], accum.at[idxc.at[u]], ss[u]).wait()

            pltpu.async_copy(row_hbm.at[pl.ds(e0s[u], CHUNK)], idxr.at[u], si[u])
            pltpu.async_copy(col_hbm.at[pl.ds(e0s[u], CHUNK)], idxc.at[u], si[u])
            pltpu.async_copy(
                emlp.at[pl.ds(e0s[u], CHUNK), pl.ds(c * DH, DH)], ebuf.at[u], se[u])
        for u in range(NSLOT):
            pltpu.make_async_copy(
                row_hbm.at[pl.ds(e0s[u], CHUNK)], idxr.at[u], si[u]).wait()
            pltpu.make_async_copy(
                col_hbm.at[pl.ds(e0s[u], CHUNK)], idxc.at[u], si[u]).wait()

            @pl.when(c == 0)
            def _(u=u):
                pltpu.async_copy(pot0.at[idxc.at[u]], gbuf.at[u], sg[u])

            @pl.when(c == 1)
            def _(u=u):
                pltpu.async_copy(pot1.at[idxc.at[u]], gbuf.at[u], sg[u])

        for u in range(NSLOT):
            pltpu.make_async_copy(pot0.at[idxc.at[u]], gbuf.at[u], sg[u]).wait()
            pltpu.make_async_copy(
                emlp.at[pl.ds(e0s[u], CHUNK), pl.ds(c * DH, DH)], ebuf.at[u],
                se[u]).wait()
            pltpu.async_copy(gbuf.at[u], accum.at[idxr.at[u]], ss[u], add=True)
            pltpu.async_copy(ebuf.at[u], accum.at[idxc.at[u]], ss[u], add=True)
        return carry

    lax.fori_loop(0, NITER, edge_it, 0)
    for u in range(NSLOT):
        pltpu.make_async_copy(gbuf.at[u], accum.at[idxr.at[u]], ss[u]).wait()
        pltpu.make_async_copy(ebuf.at[u], accum.at[idxc.at[u]], ss[u]).wait()

    @pl.when(s < NLEFT)
    def _tail():
        e0 = (CPT * NS + s) * CHUNK
        pltpu.sync_copy(row_hbm.at[pl.ds(e0, CHUNK)], idxr.at[0])
        pltpu.sync_copy(col_hbm.at[pl.ds(e0, CHUNK)], idxc.at[0])

        @pl.when(c == 0)
        def _():
            pltpu.async_copy(pot0.at[idxc.at[0]], gbuf.at[0], sem).wait()

        @pl.when(c == 1)
        def _():
            pltpu.async_copy(pot1.at[idxc.at[0]], gbuf.at[0], sem).wait()

        pltpu.sync_copy(gbuf.at[0], accum.at[idxr.at[0]], add=True)
        pltpu.sync_copy(emlp.at[pl.ds(e0, CHUNK), pl.ds(c * DH, DH)], ebuf.at[0])
        pltpu.sync_copy(ebuf.at[0], accum.at[idxc.at[0]], add=True)

    plsc.subcore_barrier()

    def out_it(i, carry):
        j = i * NS + s

        @pl.when(j < NRFULL)
        def _():
            r0 = j * RCH
            pltpu.sync_copy(accum.at[pl.ds(r0, RCH)], gbuf.at[0])
            pltpu.sync_copy(gbuf.at[0], out.at[pl.ds(r0, RCH), pl.ds(c * DH, DH)])

        return carry

    lax.fori_loop(0, (NRFULL + NS - 1) // NS, out_it, 0)

    @pl.when(s == 0)
    def _():
        r0 = NRFULL * RCH
        pltpu.sync_copy(accum.at[pl.ds(r0, RTAIL)], gbuf.at[0, pl.ds(0, RTAIL)])
        pltpu.sync_copy(
            gbuf.at[0, pl.ds(0, RTAIL)],
            out.at[pl.ds(r0, RTAIL), pl.ds(c * DH, DH)],
        )


def _sc_scatter(pot0, pot1, base, emlp, row, col):
    mesh = plsc.VectorSubcoreMesh(core_axis_name="c", subcore_axis_name="s")
    k = pl.kernel(
        _sc_body,
        mesh=mesh,
        out_type=jax.ShapeDtypeStruct((N, D_OUT), jnp.float32),
        scratch_types=[
            pltpu.VMEM((NSLOT, CHUNK), jnp.int32),
            pltpu.VMEM((NSLOT, CHUNK), jnp.int32),
            pltpu.VMEM((NSLOT, CHUNK, DH), jnp.float32),
            pltpu.VMEM((NSLOT, CHUNK, DH), jnp.float32),
            pltpu.VMEM_SHARED((N, DH), jnp.float32),
        ] + [pltpu.SemaphoreType.DMA] * 13,
    )
    return k(pot0, pot1, base, emlp, row, col)


@jax.jit
def kernel(node_mat, edge_mat, e_list, node_weight, edge_lay_1, root, bias):
    pot0, pot1, base = _node_transform(
        node_mat, node_weight, root, bias.reshape(1, D_OUT)
    )
    emlp = _edge_mlp(edge_mat, edge_lay_1)
    row = e_list[0]
    col = e_list[1]
    return _sc_scatter(pot0, pot1, base, emlp, row, col)


# speedup vs baseline: 5.5378x; 1.2229x over previous
import jax
import jax.numpy as jnp
from jax import lax
from jax.experimental import pallas as pl
from jax.experimental.pallas import tpu as pltpu
from jax.experimental.pallas import tpu_sc as plsc

N = 10000
E = 160000
D_IN = 256
D_EDGE = 16
D_OUT = 256
DH = D_OUT // 2

NS = 16
CHUNK = 128
NSLOT = 3
NCHUNK = E // CHUNK
CPT = NCHUNK // NS
NLEFT = NCHUNK - CPT * NS
NITER = CPT // NSLOT
RCH = 128
NRFULL = N // RCH
RTAIL = N - NRFULL * RCH


def _node_body(x_ref, w_ref, r_ref, b_ref, p0_ref, p1_ref, base_ref):
    x = x_ref[...]
    pot = jnp.dot(x, w_ref[...], preferred_element_type=jnp.float32)
    p0_ref[...] = pot[:, :DH]
    p1_ref[...] = pot[:, DH:]
    base_ref[...] = (
        jnp.dot(x, r_ref[...], preferred_element_type=jnp.float32) + b_ref[...]
    )


def _node_transform(node_mat, node_weight, root, bias2d):
    bm = 400
    grid = (N // bm,)
    return pl.pallas_call(
        _node_body,
        grid=grid,
        in_specs=[
            pl.BlockSpec((bm, D_IN), lambda i: (i, 0)),
            pl.BlockSpec((D_IN, D_OUT), lambda i: (0, 0)),
            pl.BlockSpec((D_IN, D_OUT), lambda i: (0, 0)),
            pl.BlockSpec((1, D_OUT), lambda i: (0, 0)),
        ],
        out_specs=[
            pl.BlockSpec((bm, DH), lambda i: (i, 0)),
            pl.BlockSpec((bm, DH), lambda i: (i, 0)),
            pl.BlockSpec((bm, D_OUT), lambda i: (i, 0)),
        ],
        out_shape=[
            jax.ShapeDtypeStruct((N, DH), jnp.float32),
            jax.ShapeDtypeStruct((N, DH), jnp.float32),
            jax.ShapeDtypeStruct((N, D_OUT), jnp.float32),
        ],
    )(node_mat, node_weight, root, bias2d)


def _edge_body(e_ref, w_ref, o_ref):
    o_ref[...] = jnp.maximum(
        jnp.dot(e_ref[...], w_ref[...], preferred_element_type=jnp.float32), 0.0
    )


def _edge_mlp(edge_mat, edge_lay_1):
    bm = 1600
    grid = (E // bm,)
    return pl.pallas_call(
        _edge_body,
        grid=grid,
        in_specs=[
            pl.BlockSpec((bm, D_EDGE), lambda i: (i, 0)),
            pl.BlockSpec((D_EDGE, D_OUT), lambda i: (0, 0)),
        ],
        out_specs=pl.BlockSpec((bm, D_OUT), lambda i: (i, 0)),
        out_shape=jax.ShapeDtypeStruct((E, D_OUT), jnp.float32),
    )(edge_mat, edge_lay_1)


def _accum_init(c, s, src, accum, stage):
    def init_it(i, carry):
        j = i * NS + s

        @pl.when(j < NRFULL)
        def _():
            r0 = j * RCH
            pltpu.sync_copy(src.at[pl.ds(r0, RCH), pl.ds(c * DH, DH)], stage)
            pltpu.sync_copy(stage, accum.at[pl.ds(r0, RCH)])

        return carry

    lax.fori_loop(0, (NRFULL + NS - 1) // NS, init_it, 0)

    @pl.when(s == 0)
    def _():
        r0 = NRFULL * RCH
        pltpu.sync_copy(
            src.at[pl.ds(r0, RTAIL), pl.ds(c * DH, DH)], stage.at[pl.ds(0, RTAIL)]
        )
        pltpu.sync_copy(stage.at[pl.ds(0, RTAIL)], accum.at[pl.ds(r0, RTAIL)])


def _accum_writeout(c, s, accum, dst, stage):
    def out_it(i, carry):
        j = i * NS + s

        @pl.when(j < NRFULL)
        def _():
            r0 = j * RCH
            pltpu.sync_copy(accum.at[pl.ds(r0, RCH)], stage)
            pltpu.sync_copy(stage, dst.at[pl.ds(r0, RCH), pl.ds(c * DH, DH)])

        return carry

    lax.fori_loop(0, (NRFULL + NS - 1) // NS, out_it, 0)

    @pl.when(s == 0)
    def _():
        r0 = NRFULL * RCH
        pltpu.sync_copy(accum.at[pl.ds(r0, RTAIL)], stage.at[pl.ds(0, RTAIL)])
        pltpu.sync_copy(
            stage.at[pl.ds(0, RTAIL)], dst.at[pl.ds(r0, RTAIL), pl.ds(c * DH, DH)]
        )


def _sc_pot_body(pot0, pot1, base, row_hbm, col_hbm, out,
                 idxr, idxc, gbuf, accum,
                 si0, si1, si2, sg0, sg1, sg2, ss0, ss1, ss2):
    c = lax.axis_index("c")
    s = lax.axis_index("s")
    si = (si0, si1, si2)
    sg = (sg0, sg1, sg2)
    ss = (ss0, ss1, ss2)

    _accum_init(c, s, base, accum, gbuf.at[0])
    plsc.subcore_barrier()

    def edge_it(t, carry):
        e0s = [((t * NSLOT + u) * NS + s) * CHUNK for u in range(NSLOT)]
        for u in range(NSLOT):
            @pl.when(t > 0)
            def _(u=u):
                pltpu.make_async_copy(
                    gbuf.at[u], accum.at[idxr.at[u]], ss[u]).wait()

            pltpu.async_copy(row_hbm.at[pl.ds(e0s[u], CHUNK)], idxr.at[u], si[u])
            pltpu.async_copy(col_hbm.at[pl.ds(e0s[u], CHUNK)], idxc.at[u], si[u])
        for u in range(NSLOT):
            pltpu.make_async_copy(
                row_hbm.at[pl.ds(e0s[u], CHUNK)], idxr.at[u], si[u]).wait()
            pltpu.make_async_copy(
                col_hbm.at[pl.ds(e0s[u], CHUNK)], idxc.at[u], si[u]).wait()

            @pl.when(c == 0)
            def _(u=u):
                pltpu.async_copy(pot0.at[idxc.at[u]], gbuf.at[u], sg[u])

            @pl.when(c == 1)
            def _(u=u):
                pltpu.async_copy(pot1.at[idxc.at[u]], gbuf.at[u], sg[u])

        for u in range(NSLOT):
            pltpu.make_async_copy(pot0.at[idxc.at[u]], gbuf.at[u], sg[u]).wait()
            pltpu.async_copy(gbuf.at[u], accum.at[idxr.at[u]], ss[u], add=True)
        return carry

    lax.fori_loop(0, NITER, edge_it, 0)
    for u in range(NSLOT):
        pltpu.make_async_copy(gbuf.at[u], accum.at[idxr.at[u]], ss[u]).wait()

    @pl.when(s < NLEFT)
    def _():
        e0 = (CPT * NS + s) * CHUNK
        pltpu.sync_copy(row_hbm.at[pl.ds(e0, CHUNK)], idxr.at[0])
        pltpu.sync_copy(col_hbm.at[pl.ds(e0, CHUNK)], idxc.at[0])

        @pl.when(c == 0)
        def _():
            pltpu.async_copy(pot0.at[idxc.at[0]], gbuf.at[0], si[0]).wait()

        @pl.when(c == 1)
        def _():
            pltpu.async_copy(pot1.at[idxc.at[0]], gbuf.at[0], si[0]).wait()

        pltpu.sync_copy(gbuf.at[0], accum.at[idxr.at[0]], add=True)

    plsc.subcore_barrier()
    _accum_writeout(c, s, accum, out, gbuf.at[0])


def _sc_emlp_body(partial, emlp, col_hbm, out,
                  idxc, ebuf, accum,
                  si0, si1, si2, se0, se1, se2, ss0, ss1, ss2):
    c = lax.axis_index("c")
    s = lax.axis_index("s")
    si = (si0, si1, si2)
    se = (se0, se1, se2)
    ss = (ss0, ss1, ss2)

    _accum_init(c, s, partial, accum, ebuf.at[0])
    plsc.subcore_barrier()

    def edge_it(t, carry):
        e0s = [((t * NSLOT + u) * NS + s) * CHUNK for u in range(NSLOT)]
        for u in range(NSLOT):
            @pl.when(t > 0)
            def _(u=u):
                pltpu.make_async_copy(
                    ebuf.at[u], accum.at[idxc.at[u]], ss[u]).wait()

            pltpu.async_copy(col_hbm.at[pl.ds(e0s[u], CHUNK)], idxc.at[u], si[u])
            pltpu.async_copy(
                emlp.at[pl.ds(e0s[u], CHUNK), pl.ds(c * DH, DH)], ebuf.at[u],
                se[u])
        for u in range(NSLOT):
            pltpu.make_async_copy(
                col_hbm.at[pl.ds(e0s[u], CHUNK)], idxc.at[u], si[u]).wait()
            pltpu.make_async_copy(
                emlp.at[pl.ds(e0s[u], CHUNK), pl.ds(c * DH, DH)], ebuf.at[u],
                se[u]).wait()
            pltpu.async_copy(ebuf.at[u], accum.at[idxc.at[u]], ss[u], add=True)
        return carry

    lax.fori_loop(0, NITER, edge_it, 0)
    for u in range(NSLOT):
        pltpu.make_async_copy(ebuf.at[u], accum.at[idxc.at[u]], ss[u]).wait()

    @pl.when(s < NLEFT)
    def _():
        e0 = (CPT * NS + s) * CHUNK
        pltpu.sync_copy(col_hbm.at[pl.ds(e0, CHUNK)], idxc.at[0])
        pltpu.sync_copy(emlp.at[pl.ds(e0, CHUNK), pl.ds(c * DH, DH)], ebuf.at[0])
        pltpu.sync_copy(ebuf.at[0], accum.at[idxc.at[0]], add=True)

    plsc.subcore_barrier()
    _accum_writeout(c, s, accum, out, ebuf.at[0])


def _sc_pot_scatter(pot0, pot1, base, row, col):
    mesh = plsc.VectorSubcoreMesh(core_axis_name="c", subcore_axis_name="s")
    k = pl.kernel(
        _sc_pot_body,
        mesh=mesh,
        out_type=jax.ShapeDtypeStruct((N, D_OUT), jnp.float32),
        scratch_types=[
            pltpu.VMEM((NSLOT, CHUNK), jnp.int32),
            pltpu.VMEM((NSLOT, CHUNK), jnp.int32),
            pltpu.VMEM((NSLOT, CHUNK, DH), jnp.float32),
            pltpu.VMEM_SHARED((N, DH), jnp.float32),
        ] + [pltpu.SemaphoreType.DMA] * 9,
    )
    return k(pot0, pot1, base, row, col)


def _sc_emlp_scatter(partial, emlp, col):
    mesh = plsc.VectorSubcoreMesh(core_axis_name="c", subcore_axis_name="s")
    k = pl.kernel(
        _sc_emlp_body,
        mesh=mesh,
        out_type=jax.ShapeDtypeStruct((N, D_OUT), jnp.float32),
        scratch_types=[
            pltpu.VMEM((NSLOT, CHUNK), jnp.int32),
            pltpu.VMEM((NSLOT, CHUNK, DH), jnp.float32),
            pltpu.VMEM_SHARED((N, DH), jnp.float32),
        ] + [pltpu.SemaphoreType.DMA] * 9,
    )
    return k(partial, emlp, col)


@jax.jit
def kernel(node_mat, edge_mat, e_list, node_weight, edge_lay_1, root, bias):
    pot0, pot1, base = _node_transform(
        node_mat, node_weight, root, bias.reshape(1, D_OUT)
    )
    emlp = _edge_mlp(edge_mat, edge_lay_1)
    row = e_list[0]
    col = e_list[1]
    partial = _sc_pot_scatter(pot0, pot1, base, row, col)
    return _sc_emlp_scatter(partial, emlp, col)


# speedup vs baseline: 5.5527x; 1.0027x over previous
import jax
import jax.numpy as jnp
from jax import lax
from jax.experimental import pallas as pl
from jax.experimental.pallas import tpu as pltpu
from jax.experimental.pallas import tpu_sc as plsc

N = 10000
E = 160000
D_IN = 256
D_EDGE = 16
D_OUT = 256
DH = D_OUT // 2

NS = 16
CHUNK = 128
NSLOT = 3
NCHUNK = E // CHUNK
CPT = NCHUNK // NS
NLEFT = NCHUNK - CPT * NS
NITER = CPT // NSLOT
RCH = 128
NRFULL = N // RCH
RTAIL = N - NRFULL * RCH


def _node_body(x_ref, w_ref, r_ref, b_ref, p0_ref, p1_ref, base_ref):
    x = x_ref[...]
    pot = jnp.dot(x, w_ref[...], preferred_element_type=jnp.float32)
    p0_ref[...] = pot[:, :DH]
    p1_ref[...] = pot[:, DH:]
    base_ref[...] = (
        jnp.dot(x, r_ref[...], preferred_element_type=jnp.float32) + b_ref[...]
    )


def _node_transform(node_mat, node_weight, root, bias2d):
    bm = 400
    grid = (N // bm,)
    return pl.pallas_call(
        _node_body,
        grid=grid,
        in_specs=[
            pl.BlockSpec((bm, D_IN), lambda i: (i, 0)),
            pl.BlockSpec((D_IN, D_OUT), lambda i: (0, 0)),
            pl.BlockSpec((D_IN, D_OUT), lambda i: (0, 0)),
            pl.BlockSpec((1, D_OUT), lambda i: (0, 0)),
        ],
        out_specs=[
            pl.BlockSpec((bm, DH), lambda i: (i, 0)),
            pl.BlockSpec((bm, DH), lambda i: (i, 0)),
            pl.BlockSpec((bm, D_OUT), lambda i: (i, 0)),
        ],
        out_shape=[
            jax.ShapeDtypeStruct((N, DH), jnp.float32),
            jax.ShapeDtypeStruct((N, DH), jnp.float32),
            jax.ShapeDtypeStruct((N, D_OUT), jnp.float32),
        ],
    )(node_mat, node_weight, root, bias2d)


def _edge_body(e_ref, w_ref, o_ref):
    o_ref[...] = jnp.maximum(
        jnp.dot(e_ref[...], w_ref[...], preferred_element_type=jnp.float32), 0.0
    )


def _edge_mlp(edge_mat, edge_lay_1):
    bm = 1600
    grid = (E // bm,)
    return pl.pallas_call(
        _edge_body,
        grid=grid,
        in_specs=[
            pl.BlockSpec((bm, D_EDGE), lambda i: (i, 0)),
            pl.BlockSpec((D_EDGE, D_OUT), lambda i: (0, 0)),
        ],
        out_specs=pl.BlockSpec((bm, D_OUT), lambda i: (i, 0)),
        out_shape=jax.ShapeDtypeStruct((E, D_OUT), jnp.float32),
    )(edge_mat, edge_lay_1)


def _accum_init(c, s, src, accum, stage):
    def init_it(i, carry):
        j = i * NS + s

        @pl.when(j < NRFULL)
        def _():
            r0 = j * RCH
            pltpu.sync_copy(src.at[pl.ds(r0, RCH), pl.ds(c * DH, DH)], stage)
            pltpu.sync_copy(stage, accum.at[pl.ds(r0, RCH)])

        return carry

    lax.fori_loop(0, (NRFULL + NS - 1) // NS, init_it, 0)

    @pl.when(s == 0)
    def _():
        r0 = NRFULL * RCH
        pltpu.sync_copy(
            src.at[pl.ds(r0, RTAIL), pl.ds(c * DH, DH)], stage.at[pl.ds(0, RTAIL)]
        )
        pltpu.sync_copy(stage.at[pl.ds(0, RTAIL)], accum.at[pl.ds(r0, RTAIL)])


def _accum_writeout(c, s, accum, dst, stage):
    def out_it(i, carry):
        j = i * NS + s

        @pl.when(j < NRFULL)
        def _():
            r0 = j * RCH
            pltpu.sync_copy(accum.at[pl.ds(r0, RCH)], stage)
            pltpu.sync_copy(stage, dst.at[pl.ds(r0, RCH), pl.ds(c * DH, DH)])

        return carry

    lax.fori_loop(0, (NRFULL + NS - 1) // NS, out_it, 0)

    @pl.when(s == 0)
    def _():
        r0 = NRFULL * RCH
        pltpu.sync_copy(accum.at[pl.ds(r0, RTAIL)], stage.at[pl.ds(0, RTAIL)])
        pltpu.sync_copy(
            stage.at[pl.ds(0, RTAIL)], dst.at[pl.ds(r0, RTAIL), pl.ds(c * DH, DH)]
        )


def _sc_pot_body(pot0, pot1, base, elist_hbm, out,
                 ib, gbuf, accum,
                 si0, si1, si2, sg0, sg1, sg2, ss0, ss1, ss2):
    c = lax.axis_index("c")
    s = lax.axis_index("s")
    si = (si0, si1, si2)
    sg = (sg0, sg1, sg2)
    ss = (ss0, ss1, ss2)

    _accum_init(c, s, base, accum, gbuf.at[0])
    plsc.subcore_barrier()

    def edge_it(t, carry):
        e0s = [((t * NSLOT + u) * NS + s) * CHUNK for u in range(NSLOT)]
        for u in range(NSLOT):
            @pl.when(t > 0)
            def _(u=u):
                pltpu.make_async_copy(
                    gbuf.at[u], accum.at[ib.at[u, 0]], ss[u]).wait()

            pltpu.async_copy(
                elist_hbm.at[:, pl.ds(e0s[u], CHUNK)], ib.at[u], si[u])
        for u in range(NSLOT):
            pltpu.make_async_copy(
                elist_hbm.at[:, pl.ds(e0s[u], CHUNK)], ib.at[u], si[u]).wait()

            @pl.when(c == 0)
            def _(u=u):
                pltpu.async_copy(pot0.at[ib.at[u, 1]], gbuf.at[u], sg[u])

            @pl.when(c == 1)
            def _(u=u):
                pltpu.async_copy(pot1.at[ib.at[u, 1]], gbuf.at[u], sg[u])

        for u in range(NSLOT):
            pltpu.make_async_copy(pot0.at[ib.at[u, 1]], gbuf.at[u], sg[u]).wait()
            pltpu.async_copy(gbuf.at[u], accum.at[ib.at[u, 0]], ss[u], add=True)
        return carry

    lax.fori_loop(0, NITER, edge_it, 0)
    for u in range(NSLOT):
        pltpu.make_async_copy(gbuf.at[u], accum.at[ib.at[u, 0]], ss[u]).wait()

    @pl.when(s < NLEFT)
    def _():
        e0 = (CPT * NS + s) * CHUNK
        pltpu.sync_copy(elist_hbm.at[:, pl.ds(e0, CHUNK)], ib.at[0])

        @pl.when(c == 0)
        def _():
            pltpu.async_copy(pot0.at[ib.at[0, 1]], gbuf.at[0], si[0]).wait()

        @pl.when(c == 1)
        def _():
            pltpu.async_copy(pot1.at[ib.at[0, 1]], gbuf.at[0], si[0]).wait()

        pltpu.sync_copy(gbuf.at[0], accum.at[ib.at[0, 0]], add=True)

    plsc.subcore_barrier()
    _accum_writeout(c, s, accum, out, gbuf.at[0])


def _sc_emlp_body(partial, emlp, col_hbm, out,
                  idxc, ebuf, accum,
                  si0, si1, si2, se0, se1, se2, ss0, ss1, ss2):
    c = lax.axis_index("c")
    s = lax.axis_index("s")
    si = (si0, si1, si2)
    se = (se0, se1, se2)
    ss = (ss0, ss1, ss2)

    _accum_init(c, s, partial, accum, ebuf.at[0])
    plsc.subcore_barrier()

    def edge_it(t, carry):
        e0s = [((t * NSLOT + u) * NS + s) * CHUNK for u in range(NSLOT)]
        for u in range(NSLOT):
            @pl.when(t > 0)
            def _(u=u):
                pltpu.make_async_copy(
                    ebuf.at[u], accum.at[idxc.at[u]], ss[u]).wait()

            pltpu.async_copy(col_hbm.at[pl.ds(e0s[u], CHUNK)], idxc.at[u], si[u])
            pltpu.async_copy(
                emlp.at[pl.ds(e0s[u], CHUNK), pl.ds(c * DH, DH)], ebuf.at[u],
                se[u])
        for u in range(NSLOT):
            pltpu.make_async_copy(
                col_hbm.at[pl.ds(e0s[u], CHUNK)], idxc.at[u], si[u]).wait()
            pltpu.make_async_copy(
                emlp.at[pl.ds(e0s[u], CHUNK), pl.ds(c * DH, DH)], ebuf.at[u],
                se[u]).wait()
            pltpu.async_copy(ebuf.at[u], accum.at[idxc.at[u]], ss[u], add=True)
        return carry

    lax.fori_loop(0, NITER, edge_it, 0)
    for u in range(NSLOT):
        pltpu.make_async_copy(ebuf.at[u], accum.at[idxc.at[u]], ss[u]).wait()

    @pl.when(s < NLEFT)
    def _():
        e0 = (CPT * NS + s) * CHUNK
        pltpu.sync_copy(col_hbm.at[pl.ds(e0, CHUNK)], idxc.at[0])
        pltpu.sync_copy(emlp.at[pl.ds(e0, CHUNK), pl.ds(c * DH, DH)], ebuf.at[0])
        pltpu.sync_copy(ebuf.at[0], accum.at[idxc.at[0]], add=True)

    plsc.subcore_barrier()
    _accum_writeout(c, s, accum, out, ebuf.at[0])


def _sc_pot_scatter(pot0, pot1, base, e_list):
    mesh = plsc.VectorSubcoreMesh(core_axis_name="c", subcore_axis_name="s")
    k = pl.kernel(
        _sc_pot_body,
        mesh=mesh,
        out_type=jax.ShapeDtypeStruct((N, D_OUT), jnp.float32),
        scratch_types=[
            pltpu.VMEM((NSLOT, 2, CHUNK), jnp.int32),
            pltpu.VMEM((NSLOT, CHUNK, DH), jnp.float32),
            pltpu.VMEM_SHARED((N, DH), jnp.float32),
        ] + [pltpu.SemaphoreType.DMA] * 9,
    )
    return k(pot0, pot1, base, e_list)


def _sc_emlp_scatter(partial, emlp, col):
    mesh = plsc.VectorSubcoreMesh(core_axis_name="c", subcore_axis_name="s")
    k = pl.kernel(
        _sc_emlp_body,
        mesh=mesh,
        out_type=jax.ShapeDtypeStruct((N, D_OUT), jnp.float32),
        scratch_types=[
            pltpu.VMEM((NSLOT, CHUNK), jnp.int32),
            pltpu.VMEM((NSLOT, CHUNK, DH), jnp.float32),
            pltpu.VMEM_SHARED((N, DH), jnp.float32),
        ] + [pltpu.SemaphoreType.DMA] * 9,
    )
    return k(partial, emlp, col)


@jax.jit
def kernel(node_mat, edge_mat, e_list, node_weight, edge_lay_1, root, bias):
    pot0, pot1, base = _node_transform(
        node_mat, node_weight, root, bias.reshape(1, D_OUT)
    )
    emlp = _edge_mlp(edge_mat, edge_lay_1)
    col = e_list[1]
    partial = _sc_pot_scatter(pot0, pot1, base, e_list)
    return _sc_emlp_scatter(partial, emlp, col)


# speedup vs baseline: 5.6534x; 1.0181x over previous
import jax
import jax.numpy as jnp
from jax import lax
from jax.experimental import pallas as pl
from jax.experimental.pallas import tpu as pltpu
from jax.experimental.pallas import tpu_sc as plsc

N = 10000
E = 160000
D_IN = 256
D_EDGE = 16
D_OUT = 256
DH = D_OUT // 2

NS = 16
CHUNK = 80
NSLOT = 4
NCHUNK = E // CHUNK
CPT = NCHUNK // NS
NLEFT = NCHUNK - CPT * NS
NITER = CPT // NSLOT
TPT = CPT - NITER * NSLOT
RCH = CHUNK
NRFULL = N // RCH
RTAIL = N - NRFULL * RCH


def _node_body(x_ref, w_ref, r_ref, b_ref, p0_ref, p1_ref, base_ref):
    x = x_ref[...]
    pot = jnp.dot(x, w_ref[...], preferred_element_type=jnp.float32)
    p0_ref[...] = pot[:, :DH]
    p1_ref[...] = pot[:, DH:]
    base_ref[...] = (
        jnp.dot(x, r_ref[...], preferred_element_type=jnp.float32) + b_ref[...]
    )


def _node_transform(node_mat, node_weight, root, bias2d):
    bm = 400
    grid = (N // bm,)
    return pl.pallas_call(
        _node_body,
        grid=grid,
        in_specs=[
            pl.BlockSpec((bm, D_IN), lambda i: (i, 0)),
            pl.BlockSpec((D_IN, D_OUT), lambda i: (0, 0)),
            pl.BlockSpec((D_IN, D_OUT), lambda i: (0, 0)),
            pl.BlockSpec((1, D_OUT), lambda i: (0, 0)),
        ],
        out_specs=[
            pl.BlockSpec((bm, DH), lambda i: (i, 0)),
            pl.BlockSpec((bm, DH), lambda i: (i, 0)),
            pl.BlockSpec((bm, D_OUT), lambda i: (i, 0)),
        ],
        out_shape=[
            jax.ShapeDtypeStruct((N, DH), jnp.float32),
            jax.ShapeDtypeStruct((N, DH), jnp.float32),
            jax.ShapeDtypeStruct((N, D_OUT), jnp.float32),
        ],
    )(node_mat, node_weight, root, bias2d)


def _edge_body(e_ref, w_ref, o_ref):
    o_ref[...] = jnp.maximum(
        jnp.dot(e_ref[...], w_ref[...], preferred_element_type=jnp.float32), 0.0
    )


def _edge_mlp(edge_mat, edge_lay_1):
    bm = 1600
    grid = (E // bm,)
    return pl.pallas_call(
        _edge_body,
        grid=grid,
        in_specs=[
            pl.BlockSpec((bm, D_EDGE), lambda i: (i, 0)),
            pl.BlockSpec((D_EDGE, D_OUT), lambda i: (0, 0)),
        ],
        out_specs=pl.BlockSpec((bm, D_OUT), lambda i: (i, 0)),
        out_shape=jax.ShapeDtypeStruct((E, D_OUT), jnp.float32),
    )(edge_mat, edge_lay_1)


def _accum_init(c, s, src, accum, stage):
    def init_it(i, carry):
        j = i * NS + s

        @pl.when(j < NRFULL)
        def _():
            r0 = j * RCH
            pltpu.sync_copy(src.at[pl.ds(r0, RCH), pl.ds(c * DH, DH)], stage)
            pltpu.sync_copy(stage, accum.at[pl.ds(r0, RCH)])

        return carry

    lax.fori_loop(0, (NRFULL + NS - 1) // NS, init_it, 0)

    if RTAIL:
        @pl.when(s == 0)
        def _():
            r0 = NRFULL * RCH
            pltpu.sync_copy(
                src.at[pl.ds(r0, RTAIL), pl.ds(c * DH, DH)],
                stage.at[pl.ds(0, RTAIL)],
            )
            pltpu.sync_copy(stage.at[pl.ds(0, RTAIL)], accum.at[pl.ds(r0, RTAIL)])


def _accum_writeout(c, s, accum, dst, stage):
    def out_it(i, carry):
        j = i * NS + s

        @pl.when(j < NRFULL)
        def _():
            r0 = j * RCH
            pltpu.sync_copy(accum.at[pl.ds(r0, RCH)], stage)
            pltpu.sync_copy(stage, dst.at[pl.ds(r0, RCH), pl.ds(c * DH, DH)])

        return carry

    lax.fori_loop(0, (NRFULL + NS - 1) // NS, out_it, 0)

    if RTAIL:
        @pl.when(s == 0)
        def _():
            r0 = NRFULL * RCH
            pltpu.sync_copy(accum.at[pl.ds(r0, RTAIL)], stage.at[pl.ds(0, RTAIL)])
            pltpu.sync_copy(
                stage.at[pl.ds(0, RTAIL)],
                dst.at[pl.ds(r0, RTAIL), pl.ds(c * DH, DH)],
            )


def _sc_pot_body(pot0, pot1, base, row_hbm, col_hbm, out,
                 ib, gbuf, accum,
                 si0, si1, si2, si3, sg0, sg1, sg2, sg3,
                 ss0, ss1, ss2, ss3):
    c = lax.axis_index("c")
    s = lax.axis_index("s")
    si = (si0, si1, si2, si3)
    sg = (sg0, sg1, sg2, sg3)
    ss = (ss0, ss1, ss2, ss3)

    _accum_init(c, s, base, accum, gbuf.at[0])
    plsc.subcore_barrier()

    def edge_it(t, carry):
        e0s = [((t * NSLOT + u) * NS + s) * CHUNK for u in range(NSLOT)]
        for u in range(NSLOT):
            @pl.when(t > 0)
            def _(u=u):
                pltpu.make_async_copy(
                    gbuf.at[u], accum.at[ib.at[u, 0]], ss[u]).wait()

            pltpu.async_copy(row_hbm.at[pl.ds(e0s[u], CHUNK)], ib.at[u, 0], si[u])
            pltpu.async_copy(col_hbm.at[pl.ds(e0s[u], CHUNK)], ib.at[u, 1], si[u])
        for u in range(NSLOT):
            pltpu.make_async_copy(
                row_hbm.at[pl.ds(e0s[u], CHUNK)], ib.at[u, 0], si[u]).wait()
            pltpu.make_async_copy(
                col_hbm.at[pl.ds(e0s[u], CHUNK)], ib.at[u, 1], si[u]).wait()

            @pl.when(c == 0)
            def _(u=u):
                pltpu.async_copy(pot0.at[ib.at[u, 1]], gbuf.at[u], sg[u])

            @pl.when(c == 1)
            def _(u=u):
                pltpu.async_copy(pot1.at[ib.at[u, 1]], gbuf.at[u], sg[u])

        for u in range(NSLOT):
            pltpu.make_async_copy(pot0.at[ib.at[u, 1]], gbuf.at[u], sg[u]).wait()
            pltpu.async_copy(gbuf.at[u], accum.at[ib.at[u, 0]], ss[u], add=True)
        return carry

    lax.fori_loop(0, NITER, edge_it, 0)
    for u in range(NSLOT):
        pltpu.make_async_copy(gbuf.at[u], accum.at[ib.at[u, 0]], ss[u]).wait()

    def _pot_one(e0):
        pltpu.sync_copy(row_hbm.at[pl.ds(e0, CHUNK)], ib.at[0, 0])
        pltpu.sync_copy(col_hbm.at[pl.ds(e0, CHUNK)], ib.at[0, 1])

        @pl.when(c == 0)
        def _():
            pltpu.async_copy(pot0.at[ib.at[0, 1]], gbuf.at[0], si[0]).wait()

        @pl.when(c == 1)
        def _():
            pltpu.async_copy(pot1.at[ib.at[0, 1]], gbuf.at[0], si[0]).wait()

        pltpu.sync_copy(gbuf.at[0], accum.at[ib.at[0, 0]], add=True)

    for w in range(TPT):
        _pot_one(((NITER * NSLOT + w) * NS + s) * CHUNK)
    if NLEFT:
        @pl.when(s < NLEFT)
        def _():
            _pot_one((CPT * NS + s) * CHUNK)

    plsc.subcore_barrier()
    _accum_writeout(c, s, accum, out, gbuf.at[0])


def _sc_emlp_body(partial, emlp, col_hbm, out,
                  idxc, ebuf, accum,
                  si0, si1, si2, si3, se0, se1, se2, se3,
                  ss0, ss1, ss2, ss3):
    c = lax.axis_index("c")
    s = lax.axis_index("s")
    si = (si0, si1, si2, si3)
    se = (se0, se1, se2, se3)
    ss = (ss0, ss1, ss2, ss3)

    _accum_init(c, s, partial, accum, ebuf.at[0])
    plsc.subcore_barrier()

    def edge_it(t, carry):
        e0s = [((t * NSLOT + u) * NS + s) * CHUNK for u in range(NSLOT)]
        for u in range(NSLOT):
            @pl.when(t > 0)
            def _(u=u):
                pltpu.make_async_copy(
                    ebuf.at[u], accum.at[idxc.at[u]], ss[u]).wait()

            pltpu.async_copy(col_hbm.at[pl.ds(e0s[u], CHUNK)], idxc.at[u], si[u])
            pltpu.async_copy(
                emlp.at[pl.ds(e0s[u], CHUNK), pl.ds(c * DH, DH)], ebuf.at[u],
                se[u])
        for u in range(NSLOT):
            pltpu.make_async_copy(
                col_hbm.at[pl.ds(e0s[u], CHUNK)], idxc.at[u], si[u]).wait()
            pltpu.make_async_copy(
                emlp.at[pl.ds(e0s[u], CHUNK), pl.ds(c * DH, DH)], ebuf.at[u],
                se[u]).wait()
            pltpu.async_copy(ebuf.at[u], accum.at[idxc.at[u]], ss[u], add=True)
        return carry

    lax.fori_loop(0, NITER, edge_it, 0)
    for u in range(NSLOT):
        pltpu.make_async_copy(ebuf.at[u], accum.at[idxc.at[u]], ss[u]).wait()

    def _emlp_one(e0):
        pltpu.sync_copy(col_hbm.at[pl.ds(e0, CHUNK)], idxc.at[0])
        pltpu.sync_copy(emlp.at[pl.ds(e0, CHUNK), pl.ds(c * DH, DH)], ebuf.at[0])
        pltpu.sync_copy(ebuf.at[0], accum.at[idxc.at[0]], add=True)

    for w in range(TPT):
        _emlp_one(((NITER * NSLOT + w) * NS + s) * CHUNK)
    if NLEFT:
        @pl.when(s < NLEFT)
        def _():
            _emlp_one((CPT * NS + s) * CHUNK)

    plsc.subcore_barrier()
    _accum_writeout(c, s, accum, out, ebuf.at[0])


def _sc_pot_scatter(pot0, pot1, base, row, col):
    mesh = plsc.VectorSubcoreMesh(core_axis_name="c", subcore_axis_name="s")
    k = pl.kernel(
        _sc_pot_body,
        mesh=mesh,
        out_type=jax.ShapeDtypeStruct((N, D_OUT), jnp.float32),
        scratch_types=[
            pltpu.VMEM((NSLOT, 2, CHUNK), jnp.int32),
            pltpu.VMEM((NSLOT, CHUNK, DH), jnp.float32),
            pltpu.VMEM_SHARED((N, DH), jnp.float32),
        ] + [pltpu.SemaphoreType.DMA] * 12,
    )
    return k(pot0, pot1, base, row, col)


def _sc_emlp_scatter(partial, emlp, col):
    mesh = plsc.VectorSubcoreMesh(core_axis_name="c", subcore_axis_name="s")
    k = pl.kernel(
        _sc_emlp_body,
        mesh=mesh,
        out_type=jax.ShapeDtypeStruct((N, D_OUT), jnp.float32),
        scratch_types=[
            pltpu.VMEM((NSLOT, CHUNK), jnp.int32),
            pltpu.VMEM((NSLOT, CHUNK, DH), jnp.float32),
            pltpu.VMEM_SHARED((N, DH), jnp.float32),
        ] + [pltpu.SemaphoreType.DMA] * 12,
    )
    return k(partial, emlp, col)


@jax.jit
def kernel(node_mat, edge_mat, e_list, node_weight, edge_lay_1, root, bias):
    pot0, pot1, base = _node_transform(
        node_mat, node_weight, root, bias.reshape(1, D_OUT)
    )
    emlp = _edge_mlp(edge_mat, edge_lay_1)
    row = e_list[0]
    col = e_list[1]
    partial = _sc_pot_scatter(pot0, pot1, base, row, col)
    return _sc_emlp_scatter(partial, emlp, col)
